# Initial kernel scaffold; baseline (speedup 1.0000x reference)
import functools

import jax
import jax.numpy as jnp
from jax import lax
from jax.experimental import pallas as pl
from jax.experimental.pallas import tpu as pltpu
from jax.experimental.pallas import tpu_sc as plsc

N = 4096
C = 128
K = 53
KPOS = 3
NC = 2
NS = 16
NW = NC * NS
BPW = N // NW
NBUF = 2


def _sc_body(inputs_hbm, idx_hbm, m_hbm, out_hbm,
             inp_v, idx_v, rows0, rows1, logits_v, sem0, sem1):
    wid = lax.axis_index("s") * NC + lax.axis_index("c")
    base = wid * BPW

    pltpu.sync_copy(inputs_hbm.at[pl.ds(base, BPW), :], inp_v)
    pltpu.sync_copy(idx_hbm.at[pl.ds(base, BPW), :], idx_v)

    rows = (rows0, rows1)
    sems = (sem0, sem1)

    def issue(n, b):
        pltpu.async_copy(m_hbm.at[idx_v.at[n]], rows[b], sems[b])

    def wait(b):
        pltpu.make_async_copy(m_hbm.at[idx_v.at[0]], rows[b], sems[b]).wait()

    def compute(n, b):
        r = rows[b]
        inrow = [inp_v[n, pl.ds(16 * c, 16)] for c in range(C // 16)]
        for k in range(K):
            acc = r[k, pl.ds(0, 16)] * inrow[0]
            for c in range(1, C // 16):
                acc = acc + r[k, pl.ds(16 * c, 16)] * inrow[c]
            logits_v[n, k] = jnp.sum(acc)

    for b in range(NBUF):
        issue(b, b)

    def body(i, carry):
        for b in range(NBUF):
            n = NBUF * i + b
            wait(b)
            compute(n, b)
            issue(n + NBUF, b)
        return carry

    lax.fori_loop(0, (BPW - NBUF) // NBUF, body, 0)
    for b in range(NBUF):
        wait(b)
        compute(BPW - NBUF + b, b)

    pltpu.sync_copy(logits_v, out_hbm.at[pl.ds(base, BPW), :])


def _compute_logits(inputs, index_list, M):
    mesh = plsc.VectorSubcoreMesh(core_axis_name="c", subcore_axis_name="s")
    kern = functools.partial(
        pl.kernel,
        out_type=jax.ShapeDtypeStruct((N, K), jnp.float32),
        mesh=mesh,
        scratch_types=[
            pltpu.VMEM((BPW, C), jnp.float32),
            pltpu.VMEM((BPW, K), jnp.int32),
            pltpu.VMEM((K, C), jnp.float32),
            pltpu.VMEM((K, C), jnp.float32),
            pltpu.VMEM((BPW, K), jnp.float32),
            pltpu.SemaphoreType.DMA,
            pltpu.SemaphoreType.DMA,
        ],
    )(_sc_body)
    return kern(inputs, index_list, M)


def _loss_body(logits_ref, cof_ref, out_ref):
    l = logits_ref[...]
    m = jnp.max(l, axis=1, keepdims=True)
    s = jnp.sum(jnp.exp(l - m), axis=1, keepdims=True)
    lse = m + jnp.log(s)
    mean_lse = jnp.mean(lse)
    loss = jnp.float32(0.0)
    for j in range(KPOS):
        loss = loss + cof_ref[j] * (mean_lse - jnp.mean(l[:, j:j + 1]))
    out_ref[0, 0] = loss


def _loss(logits, cof):
    out = pl.pallas_call(
        _loss_body,
        out_shape=jax.ShapeDtypeStruct((1, 1), jnp.float32),
        in_specs=[
            pl.BlockSpec(memory_space=pltpu.VMEM),
            pl.BlockSpec(memory_space=pltpu.SMEM),
        ],
        out_specs=pl.BlockSpec(memory_space=pltpu.SMEM),
    )(logits, cof)
    return out[0, 0]


def kernel(inputs, positive_index, negative_index, cof, M):
    index_list = jnp.concatenate([positive_index, negative_index], axis=1)
    logits = _compute_logits(inputs, index_list, M)
    loss = _loss(logits, cof)
    return (loss, logits)


# speedup vs baseline: 7.4011x; 7.4011x over previous
import functools

import jax
import jax.numpy as jnp
from jax import lax
from jax.experimental import pallas as pl
from jax.experimental.pallas import tpu as pltpu
from jax.experimental.pallas import tpu_sc as plsc

N = 4096
C = 128
K = 53
KPOS = 3
NC = 2
NS = 16
NW = NC * NS
BPW = N // NW
KP = 64
NBUF = 2


def _sc_body(inputs_hbm, idx_hbm, m_hbm, out_hbm,
             inp_v, idx_v, rows0, rows1, logits_v, sem0, sem1):
    wid = lax.axis_index("s") * NC + lax.axis_index("c")
    base = wid * BPW

    pltpu.sync_copy(inputs_hbm.at[pl.ds(base, BPW), :], inp_v)
    pltpu.sync_copy(idx_hbm.at[pl.ds(base, BPW), :], idx_v)

    rows = (rows0, rows1)
    sems = (sem0, sem1)

    def issue(n, b):
        pltpu.async_copy(m_hbm.at[idx_v.at[n]], rows[b], sems[b])

    def wait(b):
        pltpu.make_async_copy(m_hbm.at[idx_v.at[0]], rows[b], sems[b]).wait()

    def compute(n, b):
        r = rows[b]
        inrow = [inp_v[n, pl.ds(16 * c, 16)] for c in range(C // 16)]
        lanes = lax.iota(jnp.int32, 16)
        for g in range(KP // 16):
            kn = min(16, K - g * 16)
            res = jnp.zeros((16,), jnp.float32)
            for j in range(kn):
                k = g * 16 + j
                acc = r[k, pl.ds(0, 16)] * inrow[0]
                for c in range(1, C // 16):
                    acc = acc + r[k, pl.ds(16 * c, 16)] * inrow[c]
                res = jnp.where(lanes == j, jnp.sum(acc), res)
            logits_v[n, pl.ds(g * 16, 16)] = res

    for b in range(NBUF):
        issue(b, b)

    def body(i, carry):
        for b in range(NBUF):
            n = NBUF * i + b
            wait(b)
            compute(n, b)
            issue(n + NBUF, b)
        return carry

    lax.fori_loop(0, (BPW - NBUF) // NBUF, body, 0)
    for b in range(NBUF):
        wait(b)
        compute(BPW - NBUF + b, b)

    pltpu.sync_copy(logits_v, out_hbm.at[pl.ds(base, BPW), :])


def _compute_logits(inputs, index_list, M):
    mesh = plsc.VectorSubcoreMesh(core_axis_name="c", subcore_axis_name="s")
    kern = functools.partial(
        pl.kernel,
        out_type=jax.ShapeDtypeStruct((N, KP), jnp.float32),
        mesh=mesh,
        compiler_params=pltpu.CompilerParams(needs_layout_passes=False),
        scratch_types=[
            pltpu.VMEM((BPW, C), jnp.float32),
            pltpu.VMEM((BPW, K), jnp.int32),
            pltpu.VMEM((K, C), jnp.float32),
            pltpu.VMEM((K, C), jnp.float32),
            pltpu.VMEM((BPW, KP), jnp.float32),
            pltpu.SemaphoreType.DMA,
            pltpu.SemaphoreType.DMA,
        ],
    )(_sc_body)
    return kern(inputs, index_list, M)


def _loss_body(logits_ref, cof_ref, out_ref):
    l = logits_ref[...]
    m = jnp.max(l, axis=1, keepdims=True)
    s = jnp.sum(jnp.exp(l - m), axis=1, keepdims=True)
    lse = m + jnp.log(s)
    mean_lse = jnp.mean(lse)
    loss = jnp.float32(0.0)
    for j in range(KPOS):
        loss = loss + cof_ref[j] * (mean_lse - jnp.mean(l[:, j:j + 1]))
    out_ref[0, 0] = loss


def _loss(logits, cof):
    out = pl.pallas_call(
        _loss_body,
        out_shape=jax.ShapeDtypeStruct((1, 1), jnp.float32),
        in_specs=[
            pl.BlockSpec(memory_space=pltpu.VMEM),
            pl.BlockSpec(memory_space=pltpu.SMEM),
        ],
        out_specs=pl.BlockSpec(memory_space=pltpu.SMEM),
    )(logits, cof)
    return out[0, 0]


def kernel(inputs, positive_index, negative_index, cof, M):
    index_list = jnp.concatenate([positive_index, negative_index], axis=1)
    logits = _compute_logits(inputs, index_list, M)[:, :K]
    loss = _loss(logits, cof)
    return (loss, logits)
